# Initial kernel scaffold; baseline (speedup 1.0000x reference)
import jax
import jax.numpy as jnp
from jax.experimental import pallas as pl


def kernel(x, edge_index, W1, b1, W2, b2):
    raise NotImplementedError("write your pallas kernel here")


# speedup vs baseline: 3.0807x; 3.0807x over previous
import functools
import jax
import jax.numpy as jnp
from jax.experimental import pallas as pl
from jax.experimental.pallas import tpu as pltpu

N = 10000
F_IN = 128
HID = 128
C = 40
ROWS = 1000


def _mm1_body(x_ref, dinv_ref, w_ref, o_ref):
    xs = x_ref[...] * dinv_ref[...]
    o_ref[...] = jnp.dot(xs, w_ref[...], preferred_element_type=jnp.float32)


def _tc_mm1(x, dinv, W1):
    return pl.pallas_call(
        _mm1_body,
        grid=(N // ROWS,),
        in_specs=[
            pl.BlockSpec((ROWS, F_IN), lambda i: (i, 0)),
            pl.BlockSpec((ROWS, 1), lambda i: (i, 0)),
            pl.BlockSpec((F_IN, HID), lambda i: (0, 0)),
        ],
        out_specs=pl.BlockSpec((ROWS, HID), lambda i: (i, 0)),
        out_shape=jax.ShapeDtypeStruct((N, HID), jnp.float32),
    )(x, dinv, W1)


def _mid_body(s1_ref, h1p_ref, dinv_ref, b1_ref, w2_ref, o_ref):
    dinv = dinv_ref[...]
    agg = dinv * (s1_ref[...] + h1p_ref[...]) + b1_ref[...]
    h = jnp.maximum(agg, 0.0) * dinv
    o_ref[...] = jnp.dot(h, w2_ref[...], preferred_element_type=jnp.float32)


def _tc_mid(S1, h1p, dinv, b1, W2):
    cpad = W2.shape[1]
    return pl.pallas_call(
        _mid_body,
        grid=(N // ROWS,),
        in_specs=[
            pl.BlockSpec((ROWS, HID), lambda i: (i, 0)),
            pl.BlockSpec((ROWS, HID), lambda i: (i, 0)),
            pl.BlockSpec((ROWS, 1), lambda i: (i, 0)),
            pl.BlockSpec((1, HID), lambda i: (0, 0)),
            pl.BlockSpec((HID, cpad), lambda i: (0, 0)),
        ],
        out_specs=pl.BlockSpec((ROWS, cpad), lambda i: (i, 0)),
        out_shape=jax.ShapeDtypeStruct((N, cpad), jnp.float32),
    )(S1, h1p, dinv, b1, W2)


def _fin_body(s2_ref, h2p_ref, dinv_ref, b2_ref, o_ref):
    o = dinv_ref[...] * (s2_ref[...] + h2p_ref[...]) + b2_ref[...]
    m = jnp.max(o, axis=1, keepdims=True)
    e = jnp.exp(o - m)
    lse = m + jnp.log(jnp.sum(e, axis=1, keepdims=True))
    o_ref[...] = o - lse


def _tc_fin(S2, h2p, dinv, b2):
    return pl.pallas_call(
        _fin_body,
        grid=(N // ROWS,),
        in_specs=[
            pl.BlockSpec((ROWS, C), lambda i: (i, 0)),
            pl.BlockSpec((ROWS, C), lambda i: (i, 0)),
            pl.BlockSpec((ROWS, 1), lambda i: (i, 0)),
            pl.BlockSpec((1, C), lambda i: (0, 0)),
        ],
        out_specs=pl.BlockSpec((ROWS, C), lambda i: (i, 0)),
        out_shape=jax.ShapeDtypeStruct((N, C), jnp.float32),
    )(S2, h2p, dinv, b2)


def kernel(x, edge_index, W1, b1, W2, b2):
    src = edge_index[0]
    dst = edge_index[1]

    deg = jnp.ones((N,), jnp.float32).at[dst].add(1.0)
    dinv = jax.lax.rsqrt(deg)[:, None]

    h1p = _tc_mm1(x, dinv, W1)
    S1 = jnp.zeros((N, HID), jnp.float32).at[dst].add(h1p[src])

    h2p = _tc_mid(S1, h1p, dinv, b1[None, :], W2)
    S2 = jnp.zeros((N, C), jnp.float32).at[dst].add(h2p[src])

    return _tc_fin(S2, h2p, dinv, b2[None, :])


# speedup vs baseline: 11.5688x; 3.7553x over previous
import functools
import jax
import jax.numpy as jnp
from jax import lax
from jax.experimental import pallas as pl
from jax.experimental.pallas import tpu as pltpu
from jax.experimental.pallas import tpu_sc as plsc

N = 10000
F_IN = 128
HID = 128
C = 40
CP = 48
ROWS = 1000

NW = 32
CH = 128
NCH = 79
EPW = CH * NCH
EP = NW * EPW
NPAD = 10240
RPS = NPAD // 16

_mesh = plsc.VectorSubcoreMesh(core_axis_name="c", subcore_axis_name="s")


def _sc_deg(dst_pad, ones_rows, zeros_d):
    @functools.partial(
        pl.kernel,
        out_type=jax.ShapeDtypeStruct((2, NPAD, 16), jnp.float32),
        mesh=_mesh,
        scratch_types=[
            pltpu.VMEM((CH,), jnp.int32),
            pltpu.VMEM((CH, 16), jnp.float32),
            pltpu.VMEM_SHARED((NPAD, 16), jnp.float32),
            pltpu.SemaphoreType.DMA,
        ],
    )
    def k(dst_hbm, ones_hbm, zeros_hbm, out_hbm, idx_v, ones_v, acc_sh, sem):
        cid = lax.axis_index("c")
        sid = lax.axis_index("s")
        w = sid * 2 + cid
        pltpu.sync_copy(zeros_hbm, acc_sh.at[pl.ds(sid * RPS, RPS)])
        pltpu.sync_copy(ones_hbm, ones_v)
        plsc.subcore_barrier()

        @pl.loop(0, NCH)
        def _(c):
            base = w * EPW + c * CH
            pltpu.sync_copy(dst_hbm.at[pl.ds(base, CH)], idx_v)
            pltpu.sync_copy(ones_v, acc_sh.at[idx_v], add=True)

        plsc.subcore_barrier()
        pltpu.sync_copy(
            acc_sh.at[pl.ds(sid * RPS, RPS)],
            out_hbm.at[cid, pl.ds(sid * RPS, RPS)],
        )

    return k(dst_pad, ones_rows, zeros_d)


def _make_sc_agg(D):
    @functools.partial(
        pl.kernel,
        out_type=jax.ShapeDtypeStruct((2, NPAD, D), jnp.float32),
        mesh=_mesh,
        compiler_params=pltpu.CompilerParams(use_tc_tiling_on_sc=False),
        scratch_types=[
            pltpu.VMEM((CH,), jnp.int32),
            pltpu.VMEM((CH,), jnp.int32),
            pltpu.VMEM((CH, D), jnp.float32),
            pltpu.VMEM_SHARED((NPAD, D), jnp.float32),
            pltpu.SemaphoreType.DMA,
        ],
    )
    def k(tab_hbm, src_hbm, dst_hbm, zeros_hbm, out_hbm,
          src_v, dst_v, rows_v, acc_sh, sem):
        cid = lax.axis_index("c")
        sid = lax.axis_index("s")
        w = sid * 2 + cid
        pltpu.sync_copy(zeros_hbm, acc_sh.at[pl.ds(sid * RPS, RPS)])
        plsc.subcore_barrier()

        @pl.loop(0, NCH)
        def _(c):
            base = w * EPW + c * CH
            pltpu.sync_copy(src_hbm.at[pl.ds(base, CH)], src_v)
            pltpu.sync_copy(dst_hbm.at[pl.ds(base, CH)], dst_v)
            pltpu.async_copy(tab_hbm.at[src_v], rows_v, sem).wait()
            pltpu.sync_copy(rows_v, acc_sh.at[dst_v], add=True)

        plsc.subcore_barrier()
        pltpu.sync_copy(
            acc_sh.at[pl.ds(sid * RPS, RPS)],
            out_hbm.at[cid, pl.ds(sid * RPS, RPS)],
        )

    return k


_sc_agg_h = _make_sc_agg(HID)
_sc_agg_c = _make_sc_agg(CP)


def _mm1_body(x_ref, degp_ref, w_ref, o_ref, dinv_ref):
    deg = 1.0 + degp_ref[0, :, 0] + degp_ref[1, :, 0]
    dinv = lax.rsqrt(deg)[:, None]
    dinv_ref[...] = dinv
    xs = x_ref[...] * dinv
    o_ref[...] = jnp.dot(xs, w_ref[...], preferred_element_type=jnp.float32)


def _tc_mm1(x, degp, W1):
    return pl.pallas_call(
        _mm1_body,
        grid=(N // ROWS,),
        in_specs=[
            pl.BlockSpec((ROWS, F_IN), lambda i: (i, 0)),
            pl.BlockSpec((2, ROWS, 16), lambda i: (0, i, 0)),
            pl.BlockSpec((F_IN, HID), lambda i: (0, 0)),
        ],
        out_specs=[
            pl.BlockSpec((ROWS, HID), lambda i: (i, 0)),
            pl.BlockSpec((ROWS, 1), lambda i: (i, 0)),
        ],
        out_shape=[
            jax.ShapeDtypeStruct((N, HID), jnp.float32),
            jax.ShapeDtypeStruct((N, 1), jnp.float32),
        ],
    )(x, degp, W1)


def _mid_body(s1_ref, h1p_ref, dinv_ref, b1_ref, w2_ref, o_ref):
    dinv = dinv_ref[...]
    agg = dinv * (s1_ref[0] + s1_ref[1] + h1p_ref[...]) + b1_ref[...]
    h = jnp.maximum(agg, 0.0) * dinv
    o_ref[...] = jnp.dot(h, w2_ref[...], preferred_element_type=jnp.float32)


def _tc_mid(S1p, h1p, dinv, b1, W2p):
    return pl.pallas_call(
        _mid_body,
        grid=(N // ROWS,),
        in_specs=[
            pl.BlockSpec((2, ROWS, HID), lambda i: (0, i, 0)),
            pl.BlockSpec((ROWS, HID), lambda i: (i, 0)),
            pl.BlockSpec((ROWS, 1), lambda i: (i, 0)),
            pl.BlockSpec((1, HID), lambda i: (0, 0)),
            pl.BlockSpec((HID, CP), lambda i: (0, 0)),
        ],
        out_specs=pl.BlockSpec((ROWS, CP), lambda i: (i, 0)),
        out_shape=jax.ShapeDtypeStruct((N, CP), jnp.float32),
    )(S1p, h1p, dinv, b1, W2p)


def _fin_body(s2_ref, h2p_ref, dinv_ref, b2_ref, o_ref):
    o = dinv_ref[...] * (
        s2_ref[0, :, :C] + s2_ref[1, :, :C] + h2p_ref[:, :C]
    ) + b2_ref[...]
    m = jnp.max(o, axis=1, keepdims=True)
    e = jnp.exp(o - m)
    lse = m + jnp.log(jnp.sum(e, axis=1, keepdims=True))
    o_ref[...] = o - lse


def _tc_fin(S2p, h2p, dinv, b2):
    return pl.pallas_call(
        _fin_body,
        grid=(N // ROWS,),
        in_specs=[
            pl.BlockSpec((2, ROWS, CP), lambda i: (0, i, 0)),
            pl.BlockSpec((ROWS, CP), lambda i: (i, 0)),
            pl.BlockSpec((ROWS, 1), lambda i: (i, 0)),
            pl.BlockSpec((1, C), lambda i: (0, 0)),
        ],
        out_specs=pl.BlockSpec((ROWS, C), lambda i: (i, 0)),
        out_shape=jax.ShapeDtypeStruct((N, C), jnp.float32),
    )(S2p, h2p, dinv, b2)


def kernel(x, edge_index, W1, b1, W2, b2):
    E = edge_index.shape[1]
    npad_e = EP - E
    src = jnp.concatenate(
        [edge_index[0], jnp.zeros((npad_e,), jnp.int32)])
    dst = jnp.concatenate(
        [edge_index[1], jnp.full((npad_e,), N, jnp.int32)])

    ones_rows = jnp.ones((CH, 16), jnp.float32)
    zeros_d = jnp.zeros((RPS, 16), jnp.float32)
    zeros_h = jnp.zeros((RPS, HID), jnp.float32)
    zeros_c = jnp.zeros((RPS, CP), jnp.float32)
    W2p = jnp.pad(W2, ((0, 0), (0, CP - C)))

    degp = _sc_deg(dst, ones_rows, zeros_d)
    h1p, dinv = _tc_mm1(x, degp, W1)
    S1p = _sc_agg_h(h1p, src, dst, zeros_h)
    h2p = _tc_mid(S1p, h1p, dinv, b1[None, :], W2p)
    S2p = _sc_agg_c(h2p, src, dst, zeros_c)
    return _tc_fin(S2p, h2p, dinv, b2[None, :])


# speedup vs baseline: 12.4743x; 1.0783x over previous
import functools
import jax
import jax.numpy as jnp
from jax import lax
from jax.experimental import pallas as pl
from jax.experimental.pallas import tpu as pltpu
from jax.experimental.pallas import tpu_sc as plsc

N = 10000
F_IN = 128
HID = 128
C = 40
CP = 48
ROWS = 1000

NW = 32
CH = 128
NCH = 80
EPW = CH * NCH
EP = NW * EPW
NCHP = 40
NPAD = 10240
RPS = NPAD // 16

_mesh = plsc.VectorSubcoreMesh(core_axis_name="c", subcore_axis_name="s")


def _sc_deg(dst_pad, ones_rows, zeros_d):
    @functools.partial(
        pl.kernel,
        out_type=jax.ShapeDtypeStruct((2, NPAD, 16), jnp.float32),
        mesh=_mesh,
        scratch_types=[
            pltpu.VMEM((NCH, CH), jnp.int32),
            pltpu.VMEM((CH, 16), jnp.float32),
            pltpu.VMEM_SHARED((NPAD, 16), jnp.float32),
            pltpu.SemaphoreType.DMA,
        ],
    )
    def k(dst_hbm, ones_hbm, zeros_hbm, out_hbm, idx_v, ones_v, acc_sh, sem):
        cid = lax.axis_index("c")
        sid = lax.axis_index("s")
        w = sid * 2 + cid
        pltpu.sync_copy(zeros_hbm, acc_sh.at[pl.ds(sid * RPS, RPS)])
        pltpu.sync_copy(ones_hbm, ones_v)
        pltpu.sync_copy(dst_hbm.at[w], idx_v)
        plsc.subcore_barrier()

        @pl.loop(0, NCH)
        def _(c):
            pltpu.sync_copy(ones_v, acc_sh.at[idx_v.at[c]], add=True)

        plsc.subcore_barrier()
        pltpu.sync_copy(
            acc_sh.at[pl.ds(sid * RPS, RPS)],
            out_hbm.at[cid, pl.ds(sid * RPS, RPS)],
        )

    return k(dst_pad, ones_rows, zeros_d)


def _make_sc_agg(D):
    @functools.partial(
        pl.kernel,
        out_type=jax.ShapeDtypeStruct((2, NPAD, D), jnp.float32),
        mesh=_mesh,
        compiler_params=pltpu.CompilerParams(use_tc_tiling_on_sc=False),
        scratch_types=[
            pltpu.VMEM((NCHP, CH), jnp.int32),
            pltpu.VMEM((NCHP, CH), jnp.int32),
            pltpu.VMEM((CH, D), jnp.float32),
            pltpu.VMEM((CH, D), jnp.float32),
            pltpu.VMEM_SHARED((NPAD, D), jnp.float32),
            pltpu.SemaphoreType.DMA,
            pltpu.SemaphoreType.DMA,
        ],
    )
    def k(tab_hbm, src_hbm, dst_hbm, zeros_hbm, out_hbm,
          src_v, dst_v, rows_a, rows_b, acc_sh, sem_a, sem_b):
        cid = lax.axis_index("c")
        sid = lax.axis_index("s")
        w = sid * 2 + cid
        pltpu.sync_copy(zeros_hbm, acc_sh.at[pl.ds(sid * RPS, RPS)])
        plsc.subcore_barrier()

        def gather(c, rows, sem):
            return pltpu.make_async_copy(tab_hbm.at[src_v.at[c]], rows, sem)

        for p in range(NCH // NCHP):
            pltpu.sync_copy(src_hbm.at[w, pl.ds(p * NCHP, NCHP)], src_v)
            pltpu.sync_copy(dst_hbm.at[w, pl.ds(p * NCHP, NCHP)], dst_v)
            gather(0, rows_a, sem_a).start()
            gather(1, rows_b, sem_b).start()

            @pl.loop(0, NCHP, step=2)
            def _(c):
                gather(c, rows_a, sem_a).wait()
                pltpu.sync_copy(rows_a, acc_sh.at[dst_v.at[c]], add=True)

                @pl.when(c + 2 < NCHP)
                def _():
                    gather(c + 2, rows_a, sem_a).start()

                gather(c + 1, rows_b, sem_b).wait()
                pltpu.sync_copy(rows_b, acc_sh.at[dst_v.at[c + 1]], add=True)

                @pl.when(c + 3 < NCHP)
                def _():
                    gather(c + 3, rows_b, sem_b).start()

        plsc.subcore_barrier()
        pltpu.sync_copy(
            acc_sh.at[pl.ds(sid * RPS, RPS)],
            out_hbm.at[cid, pl.ds(sid * RPS, RPS)],
        )

    return k


_sc_agg_h = _make_sc_agg(HID)
_sc_agg_c = _make_sc_agg(CP)


def _mm1_body(x_ref, degp_ref, w_ref, o_ref, dinv_ref):
    deg = 1.0 + degp_ref[0, :, 0] + degp_ref[1, :, 0]
    dinv = lax.rsqrt(deg)[:, None]
    dinv_ref[...] = dinv
    xs = x_ref[...] * dinv
    o_ref[...] = jnp.dot(xs, w_ref[...], preferred_element_type=jnp.float32)


def _tc_mm1(x, degp, W1):
    return pl.pallas_call(
        _mm1_body,
        grid=(N // ROWS,),
        in_specs=[
            pl.BlockSpec((ROWS, F_IN), lambda i: (i, 0)),
            pl.BlockSpec((2, ROWS, 16), lambda i: (0, i, 0)),
            pl.BlockSpec((F_IN, HID), lambda i: (0, 0)),
        ],
        out_specs=[
            pl.BlockSpec((ROWS, HID), lambda i: (i, 0)),
            pl.BlockSpec((ROWS, 1), lambda i: (i, 0)),
        ],
        out_shape=[
            jax.ShapeDtypeStruct((N, HID), jnp.float32),
            jax.ShapeDtypeStruct((N, 1), jnp.float32),
        ],
    )(x, degp, W1)


def _mid_body(s1_ref, h1p_ref, dinv_ref, b1_ref, w2_ref, o_ref):
    dinv = dinv_ref[...]
    agg = dinv * (s1_ref[0] + s1_ref[1] + h1p_ref[...]) + b1_ref[...]
    h = jnp.maximum(agg, 0.0) * dinv
    o_ref[...] = jnp.dot(h, w2_ref[...], preferred_element_type=jnp.float32)


def _tc_mid(S1p, h1p, dinv, b1, W2p):
    return pl.pallas_call(
        _mid_body,
        grid=(N // ROWS,),
        in_specs=[
            pl.BlockSpec((2, ROWS, HID), lambda i: (0, i, 0)),
            pl.BlockSpec((ROWS, HID), lambda i: (i, 0)),
            pl.BlockSpec((ROWS, 1), lambda i: (i, 0)),
            pl.BlockSpec((1, HID), lambda i: (0, 0)),
            pl.BlockSpec((HID, CP), lambda i: (0, 0)),
        ],
        out_specs=pl.BlockSpec((ROWS, CP), lambda i: (i, 0)),
        out_shape=jax.ShapeDtypeStruct((N, CP), jnp.float32),
    )(S1p, h1p, dinv, b1, W2p)


def _fin_body(s2_ref, h2p_ref, dinv_ref, b2_ref, o_ref):
    o = dinv_ref[...] * (
        s2_ref[0, :, :C] + s2_ref[1, :, :C] + h2p_ref[:, :C]
    ) + b2_ref[...]
    m = jnp.max(o, axis=1, keepdims=True)
    e = jnp.exp(o - m)
    lse = m + jnp.log(jnp.sum(e, axis=1, keepdims=True))
    o_ref[...] = o - lse


def _tc_fin(S2p, h2p, dinv, b2):
    return pl.pallas_call(
        _fin_body,
        grid=(N // ROWS,),
        in_specs=[
            pl.BlockSpec((2, ROWS, CP), lambda i: (0, i, 0)),
            pl.BlockSpec((ROWS, CP), lambda i: (i, 0)),
            pl.BlockSpec((ROWS, 1), lambda i: (i, 0)),
            pl.BlockSpec((1, C), lambda i: (0, 0)),
        ],
        out_specs=pl.BlockSpec((ROWS, C), lambda i: (i, 0)),
        out_shape=jax.ShapeDtypeStruct((N, C), jnp.float32),
    )(S2p, h2p, dinv, b2)


def kernel(x, edge_index, W1, b1, W2, b2):
    E = edge_index.shape[1]
    npad_e = EP - E
    src = jnp.concatenate(
        [edge_index[0], jnp.zeros((npad_e,), jnp.int32)]).reshape(NW, NCH, CH)
    dst = jnp.concatenate(
        [edge_index[1], jnp.full((npad_e,), N, jnp.int32)]).reshape(NW, NCH, CH)

    ones_rows = jnp.ones((CH, 16), jnp.float32)
    zeros_d = jnp.zeros((RPS, 16), jnp.float32)
    zeros_h = jnp.zeros((RPS, HID), jnp.float32)
    zeros_c = jnp.zeros((RPS, CP), jnp.float32)
    W2p = jnp.pad(W2, ((0, 0), (0, CP - C)))

    degp = _sc_deg(dst, ones_rows, zeros_d)
    h1p, dinv = _tc_mm1(x, degp, W1)
    S1p = _sc_agg_h(h1p, src, dst, zeros_h)
    h2p = _tc_mid(S1p, h1p, dinv, b1[None, :], W2p)
    S2p = _sc_agg_c(h2p, src, dst, zeros_c)
    return _tc_fin(S2p, h2p, dinv, b2[None, :])


# speedup vs baseline: 13.2420x; 1.0615x over previous
import functools
import jax
import jax.numpy as jnp
from jax import lax
from jax.experimental import pallas as pl
from jax.experimental.pallas import tpu as pltpu
from jax.experimental.pallas import tpu_sc as plsc

N = 10000
F_IN = 128
HID = 128
C = 40
CP = 48
ROWS = 1000

NW = 32
CH = 128
NCH = 79
EPW = CH * NCH
EP = NW * EPW
NCHP = 40
NPAD = 10240
RPS = NPAD // 16

_mesh = plsc.VectorSubcoreMesh(core_axis_name="c", subcore_axis_name="s")


def _sc_deg(dst_pad, ones_rows, zeros_d):
    @functools.partial(
        pl.kernel,
        out_type=jax.ShapeDtypeStruct((2, NPAD, 16), jnp.float32),
        mesh=_mesh,
        compiler_params=pltpu.CompilerParams(use_tc_tiling_on_sc=False),
        scratch_types=[
            pltpu.VMEM((CH,), jnp.int32),
            pltpu.VMEM((CH, 16), jnp.float32),
            pltpu.VMEM_SHARED((NPAD, 16), jnp.float32),
            pltpu.SemaphoreType.DMA,
        ],
    )
    def k(dst_hbm, ones_hbm, zeros_hbm, out_hbm, dvv, ones_v, acc_sh, sem):
        cid = lax.axis_index("c")
        sid = lax.axis_index("s")
        w = sid * 2 + cid
        pltpu.sync_copy(zeros_hbm, acc_sh.at[pl.ds(sid * RPS, RPS)])
        pltpu.sync_copy(ones_hbm, ones_v)
        plsc.subcore_barrier()

        @pl.loop(0, NCH)
        def _(c):
            base = w * EPW + c * CH
            pltpu.sync_copy(dst_hbm.at[pl.ds(base, CH)], dvv)
            pltpu.sync_copy(ones_v, acc_sh.at[dvv], add=True)

        plsc.subcore_barrier()
        pltpu.sync_copy(
            acc_sh.at[pl.ds(sid * RPS, RPS)],
            out_hbm.at[cid, pl.ds(sid * RPS, RPS)],
        )

    return k(dst_pad, ones_rows, zeros_d)


def _make_sc_agg(D):
    @functools.partial(
        pl.kernel,
        out_type=jax.ShapeDtypeStruct((2, NPAD, D), jnp.float32),
        mesh=_mesh,
        compiler_params=pltpu.CompilerParams(use_tc_tiling_on_sc=False),
        scratch_types=[
            pltpu.VMEM((CH,), jnp.int32),
            pltpu.VMEM((CH,), jnp.int32),
            pltpu.VMEM((CH, D), jnp.float32),
            pltpu.VMEM_SHARED((NPAD, D), jnp.float32),
            pltpu.SemaphoreType.DMA,
        ],
    )
    def k(tab_hbm, src_hbm, dst_hbm, zeros_hbm, out_hbm,
          sva, dva, rows_a, acc_sh, sem_a):
        cid = lax.axis_index("c")
        sid = lax.axis_index("s")
        w = sid * 2 + cid
        pltpu.sync_copy(zeros_hbm, acc_sh.at[pl.ds(sid * RPS, RPS)])
        plsc.subcore_barrier()

        @pl.loop(0, NCH)
        def _(c):
            base = w * EPW + c * CH
            pltpu.sync_copy(src_hbm.at[pl.ds(base, CH)], sva)
            pltpu.sync_copy(dst_hbm.at[pl.ds(base, CH)], dva)
            pltpu.async_copy(tab_hbm.at[sva], rows_a, sem_a).wait()
            pltpu.sync_copy(rows_a, acc_sh.at[dva], add=True)

        plsc.subcore_barrier()
        pltpu.sync_copy(
            acc_sh.at[pl.ds(sid * RPS, RPS)],
            out_hbm.at[cid, pl.ds(sid * RPS, RPS)],
        )

    return k


_sc_agg_h = _make_sc_agg(HID)
_sc_agg_c = _make_sc_agg(CP)


def _mm1_body(x_ref, degp_ref, w_ref, o_ref, dinv_ref):
    deg = 1.0 + degp_ref[0, :, 0] + degp_ref[1, :, 0]
    dinv = lax.rsqrt(deg)[:, None]
    dinv_ref[...] = dinv
    xs = x_ref[...] * dinv
    o_ref[...] = jnp.dot(xs, w_ref[...], preferred_element_type=jnp.float32)


def _tc_mm1(x, degp, W1):
    return pl.pallas_call(
        _mm1_body,
        grid=(N // ROWS,),
        in_specs=[
            pl.BlockSpec((ROWS, F_IN), lambda i: (i, 0)),
            pl.BlockSpec((2, ROWS, 16), lambda i: (0, i, 0)),
            pl.BlockSpec((F_IN, HID), lambda i: (0, 0)),
        ],
        out_specs=[
            pl.BlockSpec((ROWS, HID), lambda i: (i, 0)),
            pl.BlockSpec((ROWS, 1), lambda i: (i, 0)),
        ],
        out_shape=[
            jax.ShapeDtypeStruct((N, HID), jnp.float32),
            jax.ShapeDtypeStruct((N, 1), jnp.float32),
        ],
    )(x, degp, W1)


def _mid_body(s1_ref, h1p_ref, dinv_ref, b1_ref, w2_ref, o_ref):
    dinv = dinv_ref[...]
    agg = dinv * (s1_ref[0] + s1_ref[1] + h1p_ref[...]) + b1_ref[...]
    h = jnp.maximum(agg, 0.0) * dinv
    o_ref[...] = jnp.dot(h, w2_ref[...], preferred_element_type=jnp.float32)


def _tc_mid(S1p, h1p, dinv, b1, W2p):
    return pl.pallas_call(
        _mid_body,
        grid=(N // ROWS,),
        in_specs=[
            pl.BlockSpec((2, ROWS, HID), lambda i: (0, i, 0)),
            pl.BlockSpec((ROWS, HID), lambda i: (i, 0)),
            pl.BlockSpec((ROWS, 1), lambda i: (i, 0)),
            pl.BlockSpec((1, HID), lambda i: (0, 0)),
            pl.BlockSpec((HID, CP), lambda i: (0, 0)),
        ],
        out_specs=pl.BlockSpec((ROWS, CP), lambda i: (i, 0)),
        out_shape=jax.ShapeDtypeStruct((N, CP), jnp.float32),
    )(S1p, h1p, dinv, b1, W2p)


def _fin_body(s2_ref, h2p_ref, dinv_ref, b2_ref, o_ref):
    o = dinv_ref[...] * (
        s2_ref[0, :, :C] + s2_ref[1, :, :C] + h2p_ref[:, :C]
    ) + b2_ref[...]
    m = jnp.max(o, axis=1, keepdims=True)
    e = jnp.exp(o - m)
    lse = m + jnp.log(jnp.sum(e, axis=1, keepdims=True))
    o_ref[...] = o - lse


def _tc_fin(S2p, h2p, dinv, b2):
    return pl.pallas_call(
        _fin_body,
        grid=(N // ROWS,),
        in_specs=[
            pl.BlockSpec((2, ROWS, CP), lambda i: (0, i, 0)),
            pl.BlockSpec((ROWS, CP), lambda i: (i, 0)),
            pl.BlockSpec((ROWS, 1), lambda i: (i, 0)),
            pl.BlockSpec((1, C), lambda i: (0, 0)),
        ],
        out_specs=pl.BlockSpec((ROWS, C), lambda i: (i, 0)),
        out_shape=jax.ShapeDtypeStruct((N, C), jnp.float32),
    )(S2p, h2p, dinv, b2)


def kernel(x, edge_index, W1, b1, W2, b2):
    E = edge_index.shape[1]
    npad_e = EP - E
    src = jnp.concatenate(
        [edge_index[0], jnp.zeros((npad_e,), jnp.int32)])
    dst = jnp.concatenate(
        [edge_index[1], jnp.full((npad_e,), N, jnp.int32)])

    ones_rows = jnp.ones((CH, 16), jnp.float32)
    zeros_d = jnp.zeros((RPS, 16), jnp.float32)
    zeros_h = jnp.zeros((RPS, HID), jnp.float32)
    zeros_c = jnp.zeros((RPS, CP), jnp.float32)
    W2p = jnp.pad(W2, ((0, 0), (0, CP - C)))

    degp = _sc_deg(dst, ones_rows, zeros_d)
    h1p, dinv = _tc_mm1(x, degp, W1)
    S1p = _sc_agg_h(h1p, src, dst, zeros_h)
    h2p = _tc_mid(S1p, h1p, dinv, b1[None, :], W2p)
    S2p = _sc_agg_c(h2p, src, dst, zeros_c)
    return _tc_fin(S2p, h2p, dinv, b2[None, :])
